# Initial kernel scaffold; baseline (speedup 1.0000x reference)
import jax
import jax.numpy as jnp
from jax.experimental import pallas as pl


def kernel(x, Wr, W1, b1, W2, b2):
    raise NotImplementedError("write your pallas kernel here")


# speedup vs baseline: 3.8001x; 3.8001x over previous
import functools

import jax
import jax.numpy as jnp
from jax.experimental import pallas as pl
from jax.experimental.pallas import tpu as pltpu

DM = 1024
DFF = 2048
NE = 16
TBLK = 128
NT = 32
PADDED = NT * TBLK
FCH = 2
FFB = DFF // FCH


def _router_body(x_ref, wr_ref, idx_ref):
    logits = jax.lax.dot_general(
        x_ref[...], wr_ref[...], (((1,), (1,)), ((), ())),
        preferred_element_type=jnp.float32)
    m = jnp.max(logits, axis=-1, keepdims=True)
    e = jnp.exp(logits - m)
    probs = e / jnp.sum(e, axis=-1, keepdims=True)
    idx_ref[...] = jnp.argmax(probs, axis=-1).astype(jnp.int32)[None, :]


def _router(x_flat, Wr, interpret=False):
    T = x_flat.shape[0]
    return pl.pallas_call(
        _router_body,
        out_shape=jax.ShapeDtypeStruct((1, T), jnp.int32),
        interpret=interpret,
    )(x_flat, Wr)


def _gather_body(ids_ref, src_ref, out_ref):
    t = pl.program_id(0)

    def body(i, _):
        out_ref[i, :] = src_ref[ids_ref[t * TBLK + i], :]
        return 0

    jax.lax.fori_loop(0, TBLK, body, 0, unroll=8)


def _gather_rows(src, ids, interpret=False):
    n = ids.shape[0]
    grid_spec = pltpu.PrefetchScalarGridSpec(
        num_scalar_prefetch=1,
        grid=(n // TBLK,),
        in_specs=[pl.BlockSpec(src.shape, lambda t, ids: (0, 0))],
        out_specs=pl.BlockSpec((TBLK, src.shape[1]), lambda t, ids: (t, 0)),
    )
    return pl.pallas_call(
        _gather_body,
        grid_spec=grid_spec,
        out_shape=jax.ShapeDtypeStruct((n, src.shape[1]), src.dtype),
        interpret=interpret,
    )(ids, src)


def _ffn_body(te_ref, live_ref, xs_ref, w1_ref, b1_ref, w2_ref, b2_ref,
              ys_ref, acc_ref):
    t = pl.program_id(0)
    f = pl.program_id(1)

    @pl.when(live_ref[t] > 0)
    def _():
        h = jnp.dot(xs_ref[...], w1_ref[0], preferred_element_type=jnp.float32)
        h = h + b1_ref[0]
        h = h * 0.5 * (1.0 + jax.lax.erf(h * 0.7071067811865476))
        y = jnp.dot(h, w2_ref[0], preferred_element_type=jnp.float32)

        @pl.when(f == 0)
        def _():
            acc_ref[...] = y

        @pl.when(f > 0)
        def _():
            acc_ref[...] += y

        @pl.when(f == FCH - 1)
        def _():
            ys_ref[...] = acc_ref[...] + b2_ref[0]


def _ffn(xs, W1, b1, W2, b2, te, live, interpret=False):
    grid_spec = pltpu.PrefetchScalarGridSpec(
        num_scalar_prefetch=2,
        grid=(NT, FCH),
        in_specs=[
            pl.BlockSpec((TBLK, DM), lambda t, f, te, lv: (t, 0)),
            pl.BlockSpec((1, DM, FFB), lambda t, f, te, lv: (te[t], 0, f)),
            pl.BlockSpec((1, 1, FFB), lambda t, f, te, lv: (te[t], 0, f)),
            pl.BlockSpec((1, FFB, DM), lambda t, f, te, lv: (te[t], f, 0)),
            pl.BlockSpec((1, 1, DM), lambda t, f, te, lv: (te[t], 0, 0)),
        ],
        out_specs=pl.BlockSpec((TBLK, DM), lambda t, f, te, lv: (t, 0)),
        scratch_shapes=[pltpu.VMEM((TBLK, DM), jnp.float32)],
    )
    return pl.pallas_call(
        _ffn_body,
        grid_spec=grid_spec,
        out_shape=jax.ShapeDtypeStruct((PADDED, DM), jnp.float32),
        interpret=interpret,
    )(te, live, xs, W1, b1, W2, b2)


def _moe(x, Wr, W1, b1, W2, b2, interpret=False):
    B, T, D = x.shape
    x_flat = x.reshape(B * T, D)
    n_tok = B * T

    idx = _router(x_flat, Wr, interpret=interpret)[0]

    oh = (idx[:, None] == jnp.arange(NE, dtype=jnp.int32)[None, :])
    ohi = oh.astype(jnp.int32)
    cnt = jnp.sum(ohi, axis=0)
    rank = jnp.sum(jnp.cumsum(ohi, axis=0) * ohi, axis=1) - 1
    padded_cnt = ((cnt + TBLK - 1) // TBLK) * TBLK
    gstart = jnp.concatenate(
        [jnp.zeros((1,), jnp.int32),
         jnp.cumsum(padded_cnt)[:-1].astype(jnp.int32)])
    dest = (gstart[idx] + rank).astype(jnp.int32)
    rows = jnp.full((PADDED,), -1, jnp.int32).at[dest].set(
        jnp.arange(n_tok, dtype=jnp.int32))
    first = rows[::TBLK]
    live = (first >= 0).astype(jnp.int32)
    te_raw = idx[jnp.clip(first, 0, n_tok - 1)]
    last_te = jnp.max(jnp.where(live > 0, te_raw, -1)).astype(jnp.int32)
    te = jnp.where(live > 0, te_raw, last_te).astype(jnp.int32)
    rows_c = jnp.maximum(rows, 0)

    xs = _gather_rows(x_flat, rows_c, interpret=interpret)
    ys = _ffn(xs, W1, b1.reshape(NE, 1, DFF), W2, b2.reshape(NE, 1, DM),
              te, live, interpret=interpret)
    out = _gather_rows(ys, dest, interpret=interpret)
    return out.reshape(B, T, D)


def kernel(x, Wr, W1, b1, W2, b2):
    return _moe(x, Wr, W1, b1, W2, b2, interpret=False)


# speedup vs baseline: 4.9274x; 1.2967x over previous
import functools

import jax
import jax.numpy as jnp
from jax.experimental import pallas as pl
from jax.experimental.pallas import tpu as pltpu

DM = 1024
DFF = 2048
NE = 16
TBLK = 128
NT = 32
PADDED = NT * TBLK
FCH = 2
FFB = DFF // FCH


def _router_body(x_ref, wr_ref, idx_ref):
    logits = jax.lax.dot_general(
        x_ref[...], wr_ref[...], (((1,), (1,)), ((), ())),
        preferred_element_type=jnp.float32)
    m = jnp.max(logits, axis=-1, keepdims=True)
    e = jnp.exp(logits - m)
    probs = e / jnp.sum(e, axis=-1, keepdims=True)
    idx_ref[...] = jnp.argmax(probs, axis=-1).astype(jnp.int32)[None, :]


def _router(x_flat, Wr, interpret=False):
    T = x_flat.shape[0]
    return pl.pallas_call(
        _router_body,
        out_shape=jax.ShapeDtypeStruct((1, T), jnp.int32),
        interpret=interpret,
    )(x_flat, Wr)


def _gather_body(ids_ref, src_ref, out_ref):
    t = pl.program_id(0)

    def body(i, _):
        out_ref[i, :] = src_ref[ids_ref[t * TBLK + i], :]
        return 0

    jax.lax.fori_loop(0, TBLK, body, 0, unroll=8)


def _gather_rows(src, ids, interpret=False):
    n = ids.shape[0]
    grid_spec = pltpu.PrefetchScalarGridSpec(
        num_scalar_prefetch=1,
        grid=(n // TBLK,),
        in_specs=[pl.BlockSpec(src.shape, lambda t, ids: (0, 0))],
        out_specs=pl.BlockSpec((TBLK, src.shape[1]), lambda t, ids: (t, 0)),
    )
    return pl.pallas_call(
        _gather_body,
        grid_spec=grid_spec,
        out_shape=jax.ShapeDtypeStruct((n, src.shape[1]), src.dtype),
        interpret=interpret,
    )(ids, src)


def _ffn_body(te_ref, live_ref, xs_ref, w1_ref, b1_ref, w2_ref, b2_ref,
              ys_ref):
    t = pl.program_id(0)

    @pl.when(live_ref[t] > 0)
    def _():
        xb = xs_ref[...].astype(jnp.bfloat16)
        w1b = w1_ref[0].astype(jnp.bfloat16)
        h = jnp.dot(xb, w1b, preferred_element_type=jnp.float32)
        h = h + b1_ref[0]
        h = h * 0.5 * (1.0 + jax.lax.erf(h * 0.7071067811865476))
        w2b = w2_ref[0].astype(jnp.bfloat16)
        y = jnp.dot(h.astype(jnp.bfloat16), w2b,
                    preferred_element_type=jnp.float32)
        ys_ref[...] = y + b2_ref[0]


def _ffn(xs, W1, b1, W2, b2, te, live, interpret=False):
    grid_spec = pltpu.PrefetchScalarGridSpec(
        num_scalar_prefetch=2,
        grid=(NT,),
        in_specs=[
            pl.BlockSpec((TBLK, DM), lambda t, te, lv: (t, 0)),
            pl.BlockSpec((1, DM, DFF), lambda t, te, lv: (te[t], 0, 0)),
            pl.BlockSpec((1, 1, DFF), lambda t, te, lv: (te[t], 0, 0)),
            pl.BlockSpec((1, DFF, DM), lambda t, te, lv: (te[t], 0, 0)),
            pl.BlockSpec((1, 1, DM), lambda t, te, lv: (te[t], 0, 0)),
        ],
        out_specs=pl.BlockSpec((TBLK, DM), lambda t, te, lv: (t, 0)),
    )
    return pl.pallas_call(
        _ffn_body,
        grid_spec=grid_spec,
        out_shape=jax.ShapeDtypeStruct((PADDED, DM), jnp.float32),
        interpret=interpret,
    )(te, live, xs, W1, b1, W2, b2)


def _moe(x, Wr, W1, b1, W2, b2, interpret=False):
    B, T, D = x.shape
    x_flat = x.reshape(B * T, D)
    n_tok = B * T

    idx = _router(x_flat, Wr, interpret=interpret)[0]

    oh = (idx[:, None] == jnp.arange(NE, dtype=jnp.int32)[None, :])
    ohi = oh.astype(jnp.int32)
    cnt = jnp.sum(ohi, axis=0)
    rank = jnp.sum(jnp.cumsum(ohi, axis=0) * ohi, axis=1) - 1
    padded_cnt = ((cnt + TBLK - 1) // TBLK) * TBLK
    gstart = jnp.concatenate(
        [jnp.zeros((1,), jnp.int32),
         jnp.cumsum(padded_cnt)[:-1].astype(jnp.int32)])
    dest = (gstart[idx] + rank).astype(jnp.int32)
    rows = jnp.full((PADDED,), -1, jnp.int32).at[dest].set(
        jnp.arange(n_tok, dtype=jnp.int32))
    first = rows[::TBLK]
    live = (first >= 0).astype(jnp.int32)
    te_raw = idx[jnp.clip(first, 0, n_tok - 1)]
    last_te = jnp.max(jnp.where(live > 0, te_raw, -1)).astype(jnp.int32)
    te = jnp.where(live > 0, te_raw, last_te).astype(jnp.int32)
    rows_c = jnp.maximum(rows, 0)

    xs = _gather_rows(x_flat, rows_c, interpret=interpret)
    ys = _ffn(xs, W1, b1.reshape(NE, 1, DFF), W2, b2.reshape(NE, 1, DM),
              te, live, interpret=interpret)
    out = _gather_rows(ys, dest, interpret=interpret)
    return out.reshape(B, T, D)


def kernel(x, Wr, W1, b1, W2, b2):
    return _moe(x, Wr, W1, b1, W2, b2, interpret=False)
